# Initial kernel scaffold; baseline (speedup 1.0000x reference)
import jax
import jax.numpy as jnp
from jax.experimental import pallas as pl


def kernel(z, W0, b0, W1, b1):
    raise NotImplementedError("write your pallas kernel here")


# speedup vs baseline: 1.3261x; 1.3261x over previous
import jax
import jax.numpy as jnp
from jax.experimental import pallas as pl

FEAT = 64
SEQ = 1024
NS0, LPS0 = 24, 1
NS1, LPS1 = 7, 24
PERIOD = NS0 * LPS0 * NS1
BN = 32


def _seasonal_kernel(z_ref, w0_ref, b0_ref, w1_ref, b1_ref, out_ref):
    z = z_ref[...]
    p0 = jnp.dot(z, w0_ref[...], preferred_element_type=jnp.float32) + b0_ref[...]
    p1 = jnp.dot(z, w1_ref[...], preferred_element_type=jnp.float32) + b1_ref[...]
    p0 = p0.reshape(BN, NS0, FEAT)
    p1 = p1.reshape(BN, NS1, FEAT)
    tile0 = jnp.concatenate([p0] * NS1, axis=1)
    rep1 = jnp.repeat(p1, LPS1, axis=1)
    base = tile0 + rep1
    nfull = SEQ // PERIOD
    for i in range(nfull):
        out_ref[:, i * PERIOD:(i + 1) * PERIOD, :] = base
    tail = SEQ - nfull * PERIOD
    if tail:
        out_ref[:, nfull * PERIOD:, :] = base[:, :tail, :]


def kernel(z, W0, b0, W1, b1):
    N, LATENT = z.shape
    W0r = W0.reshape(LATENT, FEAT, NS0).transpose(0, 2, 1).reshape(LATENT, FEAT * NS0)
    b0r = b0.reshape(FEAT, NS0).transpose(1, 0).reshape(1, FEAT * NS0)
    W1r = W1.reshape(LATENT, FEAT, NS1).transpose(0, 2, 1).reshape(LATENT, FEAT * NS1)
    b1r = b1.reshape(FEAT, NS1).transpose(1, 0).reshape(1, FEAT * NS1)

    grid = (N // BN,)
    return pl.pallas_call(
        _seasonal_kernel,
        grid=grid,
        in_specs=[
            pl.BlockSpec((BN, LATENT), lambda i: (i, 0)),
            pl.BlockSpec((LATENT, FEAT * NS0), lambda i: (0, 0)),
            pl.BlockSpec((1, FEAT * NS0), lambda i: (0, 0)),
            pl.BlockSpec((LATENT, FEAT * NS1), lambda i: (0, 0)),
            pl.BlockSpec((1, FEAT * NS1), lambda i: (0, 0)),
        ],
        out_specs=pl.BlockSpec((BN, SEQ, FEAT), lambda i: (i, 0, 0)),
        out_shape=jax.ShapeDtypeStruct((N, SEQ, FEAT), jnp.float32),
    )(z, W0r, b0r, W1r, b1r)
